# baseline (speedup 1.0000x reference)
import jax
import jax.numpy as jnp
from jax.experimental import pallas as pl
from jax.experimental.pallas import tpu as pltpu
from jax.experimental.pallas import tpu_sc as plsc

_LANES = 16
_BR = 16
_BC = 256


def kernel(x, table):
    B, L, D = x.shape
    xf = x.reshape(B * L, D)
    mesh = plsc.VectorSubcoreMesh(core_axis_name="core",
                                  subcore_axis_name="subcore")

    @pl.kernel(out_type=jax.ShapeDtypeStruct((B * L, D), x.dtype), mesh=mesh)
    def sc_add(x_hbm, t_hbm, o_hbm):
        def body(x_vmem, t_vmem, o_vmem):
            @pl.loop(0, _BR)
            def _(r):
                @pl.loop(0, _BC, step=_LANES)
                def _(c):
                    s = (r, pl.ds(c, _LANES))
                    o_vmem.at[*s][...] = (
                        x_vmem.at[*s][...] + t_vmem.at[*s][...]
                    )

        n_tab_blocks = L // _BR
        pltpu.emit_pipeline(
            body,
            grid=(B * L // _BR, D // _BC),
            in_specs=[
                pl.BlockSpec((_BR, _BC), index_map=lambda i, j: (i, j)),
                pl.BlockSpec((_BR, _BC),
                             index_map=lambda i, j: (i % n_tab_blocks, j)),
            ],
            out_specs=[pl.BlockSpec((_BR, _BC), index_map=lambda i, j: (i, j))],
            core_axis_name=("core", "subcore"),
            dimension_semantics=(pltpu.PARALLEL, pltpu.PARALLEL),
        )(x_hbm, t_hbm, o_hbm)

    return sc_add(xf, table).reshape(B, L, D)


# speedup vs baseline: 5.1095x; 5.1095x over previous
import jax
import jax.numpy as jnp
from jax.experimental import pallas as pl


def _add_kernel(x_ref, t_ref, o_ref):
    i = pl.program_id(0)
    BL = x_ref.shape[1]
    o_ref[...] = x_ref[...] + t_ref[pl.ds(i * BL, BL), :]


def kernel(x, table):
    B, L, D = x.shape
    BL = 512
    return pl.pallas_call(
        _add_kernel,
        grid=(L // BL,),
        in_specs=[
            pl.BlockSpec((B, BL, D), lambda i: (0, i, 0)),
            pl.BlockSpec((L, D), lambda i: (0, 0)),
        ],
        out_specs=pl.BlockSpec((B, BL, D), lambda i: (0, i, 0)),
        out_shape=jax.ShapeDtypeStruct(x.shape, x.dtype),
    )(x, table)
